# Initial kernel scaffold; baseline (speedup 1.0000x reference)
import jax
import jax.numpy as jnp
from jax.experimental import pallas as pl


def kernel(nodes, problemClass, edge_index, edge_type, graph_ids, edgeW, edgeB, gru_Wih, gru_Whh, gru_bih, gru_bhh, fc1W, fc1b, fc2W, fc2b, fcLW, fcLb):
    raise NotImplementedError("write your pallas kernel here")


# speedup vs baseline: 5.5045x; 5.5045x over previous
import functools

import jax
import jax.numpy as jnp
from jax import lax
from jax.experimental import pallas as pl
from jax.experimental.pallas import tpu as pltpu
from jax.experimental.pallas import tpu_sc as plsc

PASSES = 8
T = 4
D = 150
DP = 160
G = 16
NP = 10240
CH = 128
NTILES = 32
PER_TILE_ROWS = NP // 16


def _mm_relu_body(h_ref, w_ref, b_ref, o_ref):
    acc = jnp.dot(h_ref[...], w_ref[...], preferred_element_type=jnp.float32)
    o_ref[...] = jnp.maximum(acc + b_ref[...], 0.0)


def _gru_body(inc_ref, h_ref, wx_ref, uh_ref, bi_ref, bh_ref, o_ref):
    x = inc_ref[0] + inc_ref[1]
    h = h_ref[...]
    gi = jnp.dot(x, wx_ref[...], preferred_element_type=jnp.float32) + bi_ref[...]
    gh = jnp.dot(h, uh_ref[...], preferred_element_type=jnp.float32) + bh_ref[...]
    r = jax.nn.sigmoid(gi[:, :DP] + gh[:, :DP])
    z = jax.nn.sigmoid(gi[:, DP:2 * DP] + gh[:, DP:2 * DP])
    n = jnp.tanh(gi[:, 2 * DP:] + r * gh[:, 2 * DP:])
    o_ref[...] = (1.0 - z) * n + z * h


def _leaky(x):
    return jnp.where(x > 0, x, 0.01 * x)


def _readout_body(h_ref, gid_ref, pc_ref, w1_ref, b1_ref, w2_ref, b2_ref,
                  wl_ref, bl_ref, o_ref, acc_ref, *, nsteps):
    i = pl.program_id(0)

    @pl.when(i == 0)
    def _():
        acc_ref[...] = jnp.zeros_like(acc_ref)

    gid = gid_ref[...]
    onehot = (gid == lax.broadcasted_iota(jnp.int32, (1, G), 1)).astype(jnp.float32)
    acc_ref[...] += lax.dot_general(onehot, h_ref[...], (((0,), (0,)), ((), ())),
                                    preferred_element_type=jnp.float32)

    @pl.when(i == nsteps - 1)
    def _():
        g = acc_ref[...]
        gl = jnp.log(g)
        gl = jnp.where(jnp.isnan(gl), 0.0, gl)
        gl = jnp.maximum(gl, 0.0)
        col = lax.broadcasted_iota(jnp.int32, (G, DP), 1)
        xin = jnp.where(col == D, pc_ref[...], gl)
        x1 = _leaky(jnp.dot(xin, w1_ref[...], preferred_element_type=jnp.float32) + b1_ref[...])
        x2 = _leaky(jnp.dot(x1, w2_ref[...], preferred_element_type=jnp.float32) + b2_ref[...])
        o_ref[...] = jnp.dot(x2, wl_ref[...], preferred_element_type=jnp.float32) + bl_ref[...]


GSZ = 8


def _make_sc_gather_scatter(nchunk):
    mesh = plsc.VectorSubcoreMesh(core_axis_name="c", subcore_axis_name="s")
    ngroup = nchunk // GSZ

    @functools.partial(
        pl.kernel,
        mesh=mesh,
        compiler_params=pltpu.CompilerParams(use_tc_tiling_on_sc=False),
        out_type=jax.ShapeDtypeStruct((2, NP, DP), jnp.float32),
        scratch_types=[
            pltpu.VMEM((GSZ, CH), jnp.int32),
            pltpu.VMEM((GSZ, CH), jnp.int32),
            pltpu.VMEM((CH, DP), jnp.float32),
            pltpu.VMEM_SHARED((NP, DP), jnp.float32),
            pltpu.SemaphoreType.DMA,
        ],
    )
    def sc_kernel(cidx_hbm, dst_hbm, msg_hbm, zeros_hbm, out_hbm,
                  idx_v, dst_v, rows_v, acc_sh, sem):
        c = lax.axis_index("c")
        s = lax.axis_index("s")
        pltpu.sync_copy(zeros_hbm, acc_sh.at[pl.ds(s * PER_TILE_ROWS, PER_TILE_ROWS)])
        plsc.subcore_barrier()

        def group(g, carry):
            pltpu.sync_copy(cidx_hbm.at[c, s, pl.ds(g * GSZ, GSZ)], idx_v)
            pltpu.sync_copy(dst_hbm.at[c, s, pl.ds(g * GSZ, GSZ)], dst_v)

            def chunk(j, carry2):
                pltpu.async_copy(msg_hbm.at[idx_v.at[j]], rows_v, sem).wait()
                pltpu.sync_copy(rows_v, acc_sh.at[dst_v.at[j]], add=True)
                return carry2

            return lax.fori_loop(0, GSZ, chunk, carry)

        lax.fori_loop(0, ngroup, group, 0)
        plsc.subcore_barrier()
        pltpu.sync_copy(acc_sh.at[pl.ds(s * PER_TILE_ROWS, PER_TILE_ROWS)],
                        out_hbm.at[c, pl.ds(s * PER_TILE_ROWS, PER_TILE_ROWS)])

    return sc_kernel


def kernel(nodes, problemClass, edge_index, edge_type, graph_ids, edgeW, edgeB,
           gru_Wih, gru_Whh, gru_bih, gru_bhh, fc1W, fc1b, fc2W, fc2b, fcLW, fcLb):
    N = nodes.shape[0]
    E = edge_index.shape[1]
    BLK = 512
    nsteps = NP // BLK

    h0 = jnp.pad(nodes, ((0, NP - N), (0, DP - D)))
    w = jnp.pad(edgeW, ((0, 0), (0, DP - D), (0, DP - D)))
    W_all = jnp.transpose(w, (2, 0, 1)).reshape(DP, T * DP)
    b_all = jnp.pad(edgeB, ((0, 0), (0, DP - D))).reshape(1, T * DP)

    def _gate_pack(m):
        m3 = m.reshape(3, D, D)
        m3 = jnp.transpose(m3, (2, 0, 1))
        m3 = jnp.pad(m3, ((0, DP - D), (0, 0), (0, DP - D)))
        return m3.reshape(DP, 3 * DP)

    Wx = _gate_pack(gru_Wih)
    Uh = _gate_pack(gru_Whh)
    bi = jnp.pad(gru_bih.reshape(3, D), ((0, 0), (0, DP - D))).reshape(1, 3 * DP)
    bh = jnp.pad(gru_bhh.reshape(3, D), ((0, 0), (0, DP - D))).reshape(1, 3 * DP)

    nchunk = -(-E // (NTILES * CH))
    nchunk = -(-nchunk // GSZ) * GSZ
    EP = NTILES * CH * nchunk
    src = edge_index[0]
    dst = edge_index[1]
    comb = src * T + edge_type
    comb = jnp.pad(comb, (0, EP - E), constant_values=N * T)
    dstp = jnp.pad(dst, (0, EP - E), constant_values=N)
    cidx3 = comb.reshape(2, 16, nchunk, CH)
    dst3 = dstp.reshape(2, 16, nchunk, CH)
    zeros_hbm = jnp.zeros((PER_TILE_ROWS, DP), jnp.float32)

    sc_gather_scatter = _make_sc_gather_scatter(nchunk)

    mm_relu = pl.pallas_call(
        _mm_relu_body,
        grid=(nsteps,),
        in_specs=[pl.BlockSpec((BLK, DP), lambda i: (i, 0)),
                  pl.BlockSpec((DP, T * DP), lambda i: (0, 0)),
                  pl.BlockSpec((1, T * DP), lambda i: (0, 0))],
        out_specs=pl.BlockSpec((BLK, T * DP), lambda i: (i, 0)),
        out_shape=jax.ShapeDtypeStruct((NP, T * DP), jnp.float32),
    )

    gru = pl.pallas_call(
        _gru_body,
        grid=(nsteps,),
        in_specs=[pl.BlockSpec((2, BLK, DP), lambda i: (0, i, 0)),
                  pl.BlockSpec((BLK, DP), lambda i: (i, 0)),
                  pl.BlockSpec((DP, 3 * DP), lambda i: (0, 0)),
                  pl.BlockSpec((DP, 3 * DP), lambda i: (0, 0)),
                  pl.BlockSpec((1, 3 * DP), lambda i: (0, 0)),
                  pl.BlockSpec((1, 3 * DP), lambda i: (0, 0))],
        out_specs=pl.BlockSpec((BLK, DP), lambda i: (i, 0)),
        out_shape=jax.ShapeDtypeStruct((NP, DP), jnp.float32),
    )

    def one_pass(_, h):
        per_type = mm_relu(h, W_all, b_all)
        msg_table = per_type.reshape(NP * T, DP)
        inc = sc_gather_scatter(cidx3, dst3, msg_table, zeros_hbm)
        return gru(inc, h, Wx, Uh, bi, bh)

    h = lax.fori_loop(0, PASSES, one_pass, h0)

    gid = jnp.pad(graph_ids, (0, NP - N), constant_values=G).reshape(NP, 1)
    w1 = jnp.pad(fc1W.T, ((0, DP - (D + 1)), (0, 0)))
    b1 = fc1b.reshape(1, 80)
    w2 = fc2W.T
    b2 = fc2b.reshape(1, 80)
    wl = fcLW.T
    bl = fcLb.reshape(1, 10)

    out = pl.pallas_call(
        functools.partial(_readout_body, nsteps=nsteps),
        grid=(nsteps,),
        in_specs=[pl.BlockSpec((BLK, DP), lambda i: (i, 0)),
                  pl.BlockSpec((BLK, 1), lambda i: (i, 0)),
                  pl.BlockSpec((G, 1), lambda i: (0, 0)),
                  pl.BlockSpec((DP, 80), lambda i: (0, 0)),
                  pl.BlockSpec((1, 80), lambda i: (0, 0)),
                  pl.BlockSpec((80, 80), lambda i: (0, 0)),
                  pl.BlockSpec((1, 80), lambda i: (0, 0)),
                  pl.BlockSpec((80, 10), lambda i: (0, 0)),
                  pl.BlockSpec((1, 10), lambda i: (0, 0))],
        out_specs=pl.BlockSpec((G, 10), lambda i: (0, 0)),
        out_shape=jax.ShapeDtypeStruct((G, 10), jnp.float32),
        scratch_shapes=[pltpu.VMEM((G, DP), jnp.float32)],
    )(h, gid, problemClass, w1, b1, w2, b2, wl, bl)
    return out
